# baseline (speedup 1.0000x reference)
import jax
import jax.numpy as jnp
from jax import lax
from jax.experimental import pallas as pl
from jax.experimental.pallas import tpu as pltpu
from jax.experimental.pallas import tpu_sc as plsc

NC, NS = 2, 16
NW = NC * NS
GSZ = 128
K = 4
CHUNK = K * GSZ


def _embed_body(idx_hbm, table_hbm, out_hbm, idx_v, rows_v, sem):
    wid = lax.axis_index("s") * NC + lax.axis_index("c")
    n_idx_rows = idx_hbm.shape[0] // NW
    steps = n_idx_rows // K
    row0 = wid * n_idx_rows

    def step(g, carry):
        r = row0 + g * K
        pltpu.sync_copy(idx_hbm.at[pl.ds(r, K)], idx_v)
        cps = [
            pltpu.async_copy(
                table_hbm.at[idx_v.at[j]],
                rows_v.at[pl.ds(j * GSZ, GSZ)],
                sem,
            )
            for j in range(K)
        ]
        for cp in cps:
            cp.wait()
        pltpu.sync_copy(rows_v, out_hbm.at[pl.ds(r * GSZ, CHUNK)])
        return carry

    lax.fori_loop(0, steps, step, 0)


def kernel(x, word_embed):
    B, L = x.shape
    _, d = word_embed.shape
    idx = x.astype(jnp.int32).reshape(-1, GSZ)
    mesh = plsc.VectorSubcoreMesh(core_axis_name="c", subcore_axis_name="s")
    out = pl.kernel(
        _embed_body,
        out_type=jax.ShapeDtypeStruct((B * L, d), jnp.float32),
        mesh=mesh,
        scratch_types=[
            pltpu.VMEM((K, GSZ), jnp.int32),
            pltpu.VMEM((CHUNK, d), jnp.float32),
            pltpu.SemaphoreType.DMA,
        ],
        compiler_params=pltpu.CompilerParams(use_tc_tiling_on_sc=False),
    )(idx, word_embed)
    return out.reshape(B, L, d)


# speedup vs baseline: 1.0309x; 1.0309x over previous
import jax
import jax.numpy as jnp
from jax import lax
from jax.experimental import pallas as pl
from jax.experimental.pallas import tpu as pltpu
from jax.experimental.pallas import tpu_sc as plsc

NC, NS = 2, 16
NW = NC * NS
GSZ = 128
K = 4
CHUNK = K * GSZ
NBUF = 2


def _embed_body(idx_hbm, table_hbm, out_hbm, idx_v, rows_v, sems):
    d = table_hbm.shape[1]
    wid = lax.axis_index("s") * NC + lax.axis_index("c")
    n_idx_rows = idx_hbm.shape[0] // NW
    steps = n_idx_rows // K
    row0 = wid * n_idx_rows

    def fire(step, slot):
        r = row0 + step * K
        pltpu.sync_copy(idx_hbm.at[pl.ds(r, K)], idx_v.at[pl.ds(slot * K, K)])
        for j in range(K):
            pltpu.async_copy(
                table_hbm.at[idx_v.at[slot * K + j]],
                rows_v.at[pl.ds(slot * CHUNK + j * GSZ, GSZ)],
                sems.at[slot],
            )

    def drain(step, slot):
        r = row0 + step * K
        for j in range(K):
            pltpu.make_async_copy(
                table_hbm.at[idx_v.at[slot * K + j]],
                rows_v.at[pl.ds(slot * CHUNK + j * GSZ, GSZ)],
                sems.at[slot],
            ).wait()
        pltpu.sync_copy(
            rows_v.at[pl.ds(slot * CHUNK, CHUNK)],
            out_hbm.at[pl.ds(r * GSZ, CHUNK)],
        )

    fire(0, 0)

    def outer(g, carry):
        fire(g + 1, 1)
        drain(g, 0)
        fire(g + 2, 0)
        drain(g + 1, 1)
        return carry

    lax.fori_loop(0, (steps - NBUF) // NBUF, lambda i, c: outer(i * NBUF, c), 0)

    fire(steps - 1, 1)
    drain(steps - 2, 0)
    drain(steps - 1, 1)


def kernel(x, word_embed):
    B, L = x.shape
    _, d = word_embed.shape
    idx = x.astype(jnp.int32).reshape(-1, GSZ)
    mesh = plsc.VectorSubcoreMesh(core_axis_name="c", subcore_axis_name="s")
    out = pl.kernel(
        _embed_body,
        out_type=jax.ShapeDtypeStruct((B * L, d), jnp.float32),
        mesh=mesh,
        scratch_types=[
            pltpu.VMEM((NBUF * K, GSZ), jnp.int32),
            pltpu.VMEM((NBUF * CHUNK, d), jnp.float32),
            pltpu.SemaphoreType.DMA((NBUF,)),
        ],
        compiler_params=pltpu.CompilerParams(use_tc_tiling_on_sc=False),
    )(idx, word_embed)
    return out.reshape(B, L, d)


# speedup vs baseline: 1.0317x; 1.0007x over previous
import jax
import jax.numpy as jnp
from jax import lax
from jax.experimental import pallas as pl
from jax.experimental.pallas import tpu as pltpu
from jax.experimental.pallas import tpu_sc as plsc

NC, NS = 2, 16
NW = NC * NS
GSZ = 512
K = 1
CHUNK = K * GSZ
NBUF = 2


def _embed_body(idx_hbm, table_hbm, out_hbm, idx_v, rows_v, sems):
    d = table_hbm.shape[1]
    wid = lax.axis_index("s") * NC + lax.axis_index("c")
    n_idx_rows = idx_hbm.shape[0] // NW
    steps = n_idx_rows // K
    row0 = wid * n_idx_rows

    def fire(step, slot):
        r = row0 + step * K
        pltpu.sync_copy(idx_hbm.at[pl.ds(r, K)], idx_v.at[pl.ds(slot * K, K)])
        for j in range(K):
            pltpu.async_copy(
                table_hbm.at[idx_v.at[slot * K + j]],
                rows_v.at[pl.ds(slot * CHUNK + j * GSZ, GSZ)],
                sems.at[slot],
            )

    def drain(step, slot):
        r = row0 + step * K
        for j in range(K):
            pltpu.make_async_copy(
                table_hbm.at[idx_v.at[slot * K + j]],
                rows_v.at[pl.ds(slot * CHUNK + j * GSZ, GSZ)],
                sems.at[slot],
            ).wait()
        pltpu.sync_copy(
            rows_v.at[pl.ds(slot * CHUNK, CHUNK)],
            out_hbm.at[pl.ds(r * GSZ, CHUNK)],
        )

    fire(0, 0)

    def outer(g, carry):
        fire(g + 1, 1)
        drain(g, 0)
        fire(g + 2, 0)
        drain(g + 1, 1)
        return carry

    lax.fori_loop(0, (steps - NBUF) // NBUF, lambda i, c: outer(i * NBUF, c), 0)

    fire(steps - 1, 1)
    drain(steps - 2, 0)
    drain(steps - 1, 1)


def kernel(x, word_embed):
    B, L = x.shape
    _, d = word_embed.shape
    idx = x.astype(jnp.int32).reshape(-1, GSZ)
    mesh = plsc.VectorSubcoreMesh(core_axis_name="c", subcore_axis_name="s")
    out = pl.kernel(
        _embed_body,
        out_type=jax.ShapeDtypeStruct((B * L, d), jnp.float32),
        mesh=mesh,
        scratch_types=[
            pltpu.VMEM((NBUF * K, GSZ), jnp.int32),
            pltpu.VMEM((NBUF * CHUNK, d), jnp.float32),
            pltpu.SemaphoreType.DMA((NBUF,)),
        ],
        compiler_params=pltpu.CompilerParams(use_tc_tiling_on_sc=False),
    )(idx, word_embed)
    return out.reshape(B, L, d)


# speedup vs baseline: 1.0369x; 1.0050x over previous
import jax
import jax.numpy as jnp
from jax import lax
from jax.experimental import pallas as pl
from jax.experimental.pallas import tpu as pltpu
from jax.experimental.pallas import tpu_sc as plsc

NC, NS = 2, 16
NW = NC * NS
CH = 4
NBUF = 2


def _embed_body(idx_hbm, table_hbm, out_hbm, idx_v, rows_v, sems):
    wid = lax.axis_index("s") * NC + lax.axis_index("c")
    rows_w = idx_hbm.shape[0] // NW
    steps = rows_w // CH
    row0 = wid * rows_w

    def fire(step, slot):
        r = row0 + step * CH
        pltpu.sync_copy(idx_hbm.at[pl.ds(r, CH)], idx_v.at[slot])
        for j in range(CH):
            pltpu.async_copy(
                table_hbm.at[idx_v.at[slot, j]],
                rows_v.at[slot, j],
                sems.at[slot],
            )

    def drain(step, slot):
        r = row0 + step * CH
        for j in range(CH):
            pltpu.make_async_copy(
                table_hbm.at[idx_v.at[slot, j]],
                rows_v.at[slot, j],
                sems.at[slot],
            ).wait()
        pltpu.sync_copy(rows_v.at[slot], out_hbm.at[pl.ds(r, CH)])

    fire(0, 0)

    def outer(g, carry):
        fire(g + 1, 1)
        drain(g, 0)
        fire(g + 2, 0)
        drain(g + 1, 1)
        return carry

    lax.fori_loop(0, (steps - NBUF) // NBUF, lambda i, c: outer(i * NBUF, c), 0)

    fire(steps - 1, 1)
    drain(steps - 2, 0)
    drain(steps - 1, 1)


def kernel(x, word_embed):
    B, L = x.shape
    _, d = word_embed.shape
    idx = x.astype(jnp.int32)
    mesh = plsc.VectorSubcoreMesh(core_axis_name="c", subcore_axis_name="s")
    out = pl.kernel(
        _embed_body,
        out_type=jax.ShapeDtypeStruct((B, L, d), jnp.float32),
        mesh=mesh,
        scratch_types=[
            pltpu.VMEM((NBUF, CH, L), jnp.int32),
            pltpu.VMEM((NBUF, CH, L, d), jnp.float32),
            pltpu.SemaphoreType.DMA((NBUF,)),
        ],
        compiler_params=pltpu.CompilerParams(use_tc_tiling_on_sc=False),
    )(idx, word_embed)
    return out
